# baseline (speedup 1.0000x reference)
import math

import jax
import jax.numpy as jnp
from jax import lax
from jax.experimental import pallas as pl
from jax.experimental.pallas import tpu as pltpu
from jax.experimental.pallas import tpu_sc as plsc

D_MODEL = 64
SCALE = math.sqrt(D_MODEL)
NUM_WORKERS = 32
CHUNK = 128
LANES = 16


def _emb_body(x_hbm, lut_hbm, out_hbm, idx_v, rows_v, gsem):
    w = lax.axis_index("s") * 2 + lax.axis_index("c")
    nchunks = idx_v.shape[0]
    base = w * (nchunks * CHUNK)

    pltpu.sync_copy(x_hbm.at[w], idx_v)

    def chunk_body(j, carry):
        pltpu.async_copy(lut_hbm.at[idx_v.at[j]], rows_v, gsem).wait()

        def row_body(i, c2):
            for c in range(D_MODEL // LANES):
                sl = pl.ds(c * LANES, LANES)
                rows_v[i, sl] = rows_v[i, sl] * SCALE
            return c2

        lax.fori_loop(0, CHUNK, row_body, 0, unroll=2)
        pltpu.sync_copy(rows_v, out_hbm.at[pl.ds(base + j * CHUNK, CHUNK)])
        return carry

    lax.fori_loop(0, nchunks, chunk_body, 0)


def kernel(x, lut):
    b, s = x.shape
    total = b * s
    nchunks = total // (NUM_WORKERS * CHUNK)
    xw = x.reshape(NUM_WORKERS, nchunks, CHUNK)
    mesh = plsc.VectorSubcoreMesh(core_axis_name="c", subcore_axis_name="s")
    out = pl.kernel(
        _emb_body,
        out_type=jax.ShapeDtypeStruct((total, D_MODEL), jnp.float32),
        mesh=mesh,
        scratch_types=[
            pltpu.VMEM((nchunks, CHUNK), jnp.int32),
            pltpu.VMEM((CHUNK, D_MODEL), jnp.float32),
            pltpu.SemaphoreType.DMA,
        ],
        compiler_params=pltpu.CompilerParams(use_tc_tiling_on_sc=False),
    )(xw, lut)
    return out.reshape(b, s, D_MODEL)


# speedup vs baseline: 1.0562x; 1.0562x over previous
import math

import jax
import jax.numpy as jnp
from jax import lax
from jax.experimental import pallas as pl
from jax.experimental.pallas import tpu as pltpu
from jax.experimental.pallas import tpu_sc as plsc

D_MODEL = 64
SCALE = math.sqrt(D_MODEL)
NUM_WORKERS = 32
CHUNK = 128
LANES = 16
NBUF = 4


def _emb_body(x_hbm, lut_hbm, out_hbm, idx_v, gbuf, sbuf, gsem, ssem):
    w = lax.axis_index("s") * 2 + lax.axis_index("c")
    nchunks = idx_v.shape[0]
    ngroups = nchunks // NBUF
    base = w * (nchunks * CHUNK)

    pltpu.sync_copy(x_hbm.at[w], idx_v)

    for b in range(NBUF):
        pltpu.async_copy(lut_hbm.at[idx_v.at[b]], gbuf.at[b], gsem.at[b])

    def group_body(g, carry):
        for b in range(NBUF):
            j = g * NBUF + b
            pltpu.make_async_copy(
                lut_hbm.at[idx_v.at[j]], gbuf.at[b], gsem.at[b]).wait()

            @pl.when(g >= 1)
            def _wait_store():
                pltpu.make_async_copy(
                    sbuf.at[b], out_hbm.at[pl.ds(base, CHUNK)],
                    ssem.at[b]).wait()

            def row_body(i, c2):
                for c in range(D_MODEL // LANES):
                    sl = pl.ds(c * LANES, LANES)
                    sbuf[b, i, sl] = gbuf[b, i, sl] * SCALE
                return c2

            lax.fori_loop(0, CHUNK, row_body, 0, unroll=2)

            pltpu.async_copy(
                sbuf.at[b], out_hbm.at[pl.ds(base + j * CHUNK, CHUNK)],
                ssem.at[b])

            @pl.when(g < ngroups - 1)
            def _prefetch():
                pltpu.async_copy(
                    lut_hbm.at[idx_v.at[j + NBUF]], gbuf.at[b], gsem.at[b])
        return carry

    lax.fori_loop(0, ngroups, group_body, 0)

    for b in range(NBUF):
        pltpu.make_async_copy(
            sbuf.at[b], out_hbm.at[pl.ds(base, CHUNK)], ssem.at[b]).wait()


def kernel(x, lut):
    b, s = x.shape
    total = b * s
    nchunks = total // (NUM_WORKERS * CHUNK)
    xw = x.reshape(NUM_WORKERS, nchunks, CHUNK)
    mesh = plsc.VectorSubcoreMesh(core_axis_name="c", subcore_axis_name="s")
    out = pl.kernel(
        _emb_body,
        out_type=jax.ShapeDtypeStruct((total, D_MODEL), jnp.float32),
        mesh=mesh,
        scratch_types=[
            pltpu.VMEM((nchunks, CHUNK), jnp.int32),
            pltpu.VMEM((NBUF, CHUNK, D_MODEL), jnp.float32),
            pltpu.VMEM((NBUF, CHUNK, D_MODEL), jnp.float32),
            pltpu.SemaphoreType.DMA((NBUF,)),
            pltpu.SemaphoreType.DMA((NBUF,)),
        ],
        compiler_params=pltpu.CompilerParams(use_tc_tiling_on_sc=False),
    )(xw, lut)
    return out.reshape(b, s, D_MODEL)
